# Initial kernel scaffold; baseline (speedup 1.0000x reference)
import jax
import jax.numpy as jnp
from jax.experimental import pallas as pl


def kernel(x, W_in, W_fb, bias):
    raise NotImplementedError("write your pallas kernel here")


# speedup vs baseline: 17.6497x; 17.6497x over previous
import jax
import jax.numpy as jnp
from jax.experimental import pallas as pl
from jax.experimental.pallas import tpu as pltpu

_B, _S, _DIN, _R = 32, 4096, 8, 512
_TAPS = (1, 4, 24, 96, 168)
_NTAPS = len(_TAPS)
_MAXD = max(_TAPS)
_LEAK = 0.1
_T = 128
_NC = _S // _T


def _reservoir_body(x_ref, wint_ref, wcat_ref, bias_ref, out_ref,
                    hist_ref, drive_ref):
    c = pl.program_id(0)

    @pl.when(c == 0)
    def _init():
        hist_ref[...] = jnp.zeros_like(hist_ref)

    drive_ref[...] = jax.lax.dot_general(
        x_ref[...], wint_ref[...],
        dimension_numbers=(((2,), (0,)), ((), ())),
        preferred_element_type=jnp.float32)

    wcat = wcat_ref[...]
    bias = bias_ref[...]
    base = c * _T

    def step(t, _):
        tg = base + t
        parts = [hist_ref[jax.lax.rem(tg - tau + _MAXD, _MAXD)]
                 for tau in _TAPS]
        h_prev = parts[0]
        delayed = jnp.concatenate(parts, axis=1)
        fb = jnp.dot(delayed, wcat, preferred_element_type=jnp.float32)
        h_new = ((1.0 - _LEAK) * h_prev
                 + _LEAK * jnp.tanh(drive_ref[t] + fb + bias))
        hist_ref[jax.lax.rem(tg, _MAXD)] = h_new
        out_ref[t] = h_new
        return ()

    jax.lax.fori_loop(0, _T, step, (), unroll=False)


def _run_reservoir(xt, wint, wcat, bias2):
    return pl.pallas_call(
        _reservoir_body,
        out_shape=jax.ShapeDtypeStruct((_S, _B, _R), jnp.float32),
        grid=(_NC,),
        in_specs=[
            pl.BlockSpec((_T, _B, _DIN), lambda c: (c, 0, 0)),
            pl.BlockSpec((_DIN, _R), lambda c: (0, 0)),
            pl.BlockSpec((_NTAPS * _R, _R), lambda c: (0, 0)),
            pl.BlockSpec((1, _R), lambda c: (0, 0)),
        ],
        out_specs=pl.BlockSpec((_T, _B, _R), lambda c: (c, 0, 0)),
        scratch_shapes=[
            pltpu.VMEM((_MAXD, _B, _R), jnp.float32),
            pltpu.VMEM((_T, _B, _R), jnp.float32),
        ],
        compiler_params=pltpu.CompilerParams(
            dimension_semantics=("arbitrary",),
            vmem_limit_bytes=56 * 1024 * 1024,
        ),
        name="delay_reservoir",
    )(xt, wint, wcat, bias2)


def kernel(x, W_in, W_fb, bias):
    xt = jnp.swapaxes(x, 0, 1)
    wint = jnp.transpose(W_in)
    wcat = jnp.reshape(W_fb, (_NTAPS * _R, _R))
    bias2 = jnp.reshape(bias, (1, _R))
    states = _run_reservoir(xt, wint, wcat, bias2)
    return jnp.swapaxes(states, 0, 1)


# speedup vs baseline: 33.8164x; 1.9160x over previous
import jax
import jax.numpy as jnp
from jax.experimental import pallas as pl
from jax.experimental.pallas import tpu as pltpu

_B, _S, _DIN, _R = 32, 4096, 8, 512
_TAPS = (1, 4, 24, 96, 168)
_NTAPS = len(_TAPS)
_MAXD = max(_TAPS)
_LEAK = 0.1
_T = 128
_NC = _S // _T
_BLK = 8
_NBLK = _T // _BLK


def _dot(a, b):
    return jnp.dot(a, b, preferred_element_type=jnp.float32)


def _reservoir_body(x_ref, wint_ref, wcat_ref, bias_ref, out_ref,
                    hist_ref, drive_ref, pre_ref, pre4_ref):
    c = pl.program_id(0)

    @pl.when(c == 0)
    def _init():
        hist_ref[...] = jnp.zeros_like(hist_ref)

    drive_ref[...] = jax.lax.dot_general(
        x_ref[...], wint_ref[...],
        dimension_numbers=(((2,), (0,)), ((), ())),
        preferred_element_type=jnp.float32)

    w1 = wcat_ref[0 * _R:1 * _R, :]
    w4 = wcat_ref[1 * _R:2 * _R, :]
    w24 = wcat_ref[2 * _R:3 * _R, :]
    w96 = wcat_ref[3 * _R:4 * _R, :]
    w168 = wcat_ref[4 * _R:5 * _R, :]
    bias = bias_ref[...]
    base = c * _T

    def block(blk, h_prev):
        tg0 = base + _BLK * blk
        t0 = _BLK * blk

        a24 = hist_ref[pl.ds(jax.lax.rem(tg0 + _MAXD - 24, _MAXD), _BLK)]
        a96 = hist_ref[pl.ds(jax.lax.rem(tg0 + _MAXD - 96, _MAXD), _BLK)]
        a168 = hist_ref[pl.ds(jax.lax.rem(tg0 + _MAXD - 168, _MAXD), _BLK)]
        m8 = _BLK * _B
        p8 = (_dot(a24.reshape(m8, _R), w24)
              + _dot(a96.reshape(m8, _R), w96)
              + _dot(a168.reshape(m8, _R), w168))
        pre_ref[...] = (p8.reshape(_BLK, _B, _R)
                        + drive_ref[pl.ds(t0, _BLK)] + bias)

        for sb in range(2):
            r4 = jax.lax.rem(tg0 + 4 * sb + _MAXD - 4, _MAXD)
            a4 = hist_ref[pl.ds(r4, 4)]
            p4 = _dot(a4.reshape(4 * _B, _R), w4)
            pre4_ref[...] = (pre_ref[4 * sb:4 * sb + 4]
                             + p4.reshape(4, _B, _R))
            for s in range(4):
                tg = tg0 + 4 * sb + s
                fb = _dot(h_prev, w1)
                act = jnp.tanh(pre4_ref[s] + fb)
                h_prev = (1.0 - _LEAK) * h_prev + _LEAK * act
                hist_ref[jax.lax.rem(tg, _MAXD)] = h_prev
                out_ref[t0 + 4 * sb + s] = h_prev
        return h_prev

    h0 = hist_ref[jax.lax.rem(base + _MAXD - 1, _MAXD)]
    jax.lax.fori_loop(0, _NBLK, block, h0)


def _run_reservoir(xt, wint, wcat, bias2):
    return pl.pallas_call(
        _reservoir_body,
        out_shape=jax.ShapeDtypeStruct((_S, _B, _R), jnp.float32),
        grid=(_NC,),
        in_specs=[
            pl.BlockSpec((_T, _B, _DIN), lambda c: (c, 0, 0)),
            pl.BlockSpec((_DIN, _R), lambda c: (0, 0)),
            pl.BlockSpec((_NTAPS * _R, _R), lambda c: (0, 0)),
            pl.BlockSpec((1, _R), lambda c: (0, 0)),
        ],
        out_specs=pl.BlockSpec((_T, _B, _R), lambda c: (c, 0, 0)),
        scratch_shapes=[
            pltpu.VMEM((_MAXD, _B, _R), jnp.float32),
            pltpu.VMEM((_T, _B, _R), jnp.float32),
            pltpu.VMEM((_BLK, _B, _R), jnp.float32),
            pltpu.VMEM((4, _B, _R), jnp.float32),
        ],
        compiler_params=pltpu.CompilerParams(
            dimension_semantics=("arbitrary",),
            vmem_limit_bytes=56 * 1024 * 1024,
        ),
        name="delay_reservoir",
    )(xt, wint, wcat, bias2)


def kernel(x, W_in, W_fb, bias):
    xt = jnp.swapaxes(x, 0, 1)
    wint = jnp.transpose(W_in)
    wcat = jnp.reshape(W_fb, (_NTAPS * _R, _R))
    bias2 = jnp.reshape(bias, (1, _R))
    states = _run_reservoir(xt, wint, wcat, bias2)
    return jnp.swapaxes(states, 0, 1)


# speedup vs baseline: 33.8552x; 1.0011x over previous
import jax
import jax.numpy as jnp
from jax.experimental import pallas as pl
from jax.experimental.pallas import tpu as pltpu

_B, _S, _DIN, _R = 32, 4096, 8, 512
_NTAPS = 5
_MAXD = 168
_LEAK = 0.1
_T = 128
_NC = _S // _T
_BLK = 8
_NBLK = _T // _BLK


def _dot(a, b):
    return jnp.dot(a, b, preferred_element_type=jnp.float32)


def _rowslice(ref, slot, nslots):
    idx = pl.multiple_of(slot * _B, _B)
    return ref.at[pl.ds(idx, nslots * _B), :]


def _reservoir_body(x_ref, wint_ref, wcat_ref, bias_ref, out_ref,
                    hist_ref, drive_ref, pre_ref, pre4_ref):
    c = pl.program_id(0)

    @pl.when(c == 0)
    def _init():
        hist_ref[...] = jnp.zeros_like(hist_ref)

    drive_ref[...] = _dot(x_ref[...], wint_ref[...])

    w1 = wcat_ref[0 * _R:1 * _R, :]
    w4 = wcat_ref[1 * _R:2 * _R, :]
    w24 = wcat_ref[2 * _R:3 * _R, :]
    w96 = wcat_ref[3 * _R:4 * _R, :]
    w168 = wcat_ref[4 * _R:5 * _R, :]
    bias = bias_ref[...]
    base = c * _T

    def block(blk, h_prev):
        tg0 = base + _BLK * blk
        t0 = _BLK * blk

        a24 = _rowslice(hist_ref, jax.lax.rem(tg0 + _MAXD - 24, _MAXD), _BLK)
        a96 = _rowslice(hist_ref, jax.lax.rem(tg0 + _MAXD - 96, _MAXD), _BLK)
        a168 = _rowslice(hist_ref, jax.lax.rem(tg0 + _MAXD - 168, _MAXD), _BLK)
        p8 = _dot(a24[...], w24) + _dot(a96[...], w96) + _dot(a168[...], w168)
        pre_ref[...] = p8 + _rowslice(drive_ref, t0, _BLK)[...] + bias

        for sb in range(2):
            r4 = jax.lax.rem(tg0 + 4 * sb + _MAXD - 4, _MAXD)
            a4 = _rowslice(hist_ref, r4, 4)
            pre4_ref[...] = pre_ref[4 * sb * _B:(4 * sb + 4) * _B, :] \
                + _dot(a4[...], w4)
            for s in range(4):
                tg = tg0 + 4 * sb + s
                fb = _dot(h_prev, w1)
                act = jnp.tanh(pre4_ref[s * _B:(s + 1) * _B, :] + fb)
                h_prev = (1.0 - _LEAK) * h_prev + _LEAK * act
                _rowslice(hist_ref, jax.lax.rem(tg, _MAXD), 1)[...] = h_prev
                _rowslice(out_ref, t0 + 4 * sb + s, 1)[...] = h_prev
        return h_prev

    h0 = _rowslice(hist_ref, jax.lax.rem(base + _MAXD - 1, _MAXD), 1)[...]
    jax.lax.fori_loop(0, _NBLK, block, h0)


def _run_reservoir(xt2, wint, wcat, bias2):
    return pl.pallas_call(
        _reservoir_body,
        out_shape=jax.ShapeDtypeStruct((_S * _B, _R), jnp.float32),
        grid=(_NC,),
        in_specs=[
            pl.BlockSpec((_T * _B, _DIN), lambda c: (c, 0)),
            pl.BlockSpec((_DIN, _R), lambda c: (0, 0)),
            pl.BlockSpec((_NTAPS * _R, _R), lambda c: (0, 0)),
            pl.BlockSpec((1, _R), lambda c: (0, 0)),
        ],
        out_specs=pl.BlockSpec((_T * _B, _R), lambda c: (c, 0)),
        scratch_shapes=[
            pltpu.VMEM((_MAXD * _B, _R), jnp.float32),
            pltpu.VMEM((_T * _B, _R), jnp.float32),
            pltpu.VMEM((_BLK * _B, _R), jnp.float32),
            pltpu.VMEM((4 * _B, _R), jnp.float32),
        ],
        compiler_params=pltpu.CompilerParams(
            dimension_semantics=("arbitrary",),
            vmem_limit_bytes=56 * 1024 * 1024,
        ),
        name="delay_reservoir",
    )(xt2, wint, wcat, bias2)


def kernel(x, W_in, W_fb, bias):
    xt2 = jnp.reshape(jnp.swapaxes(x, 0, 1), (_S * _B, _DIN))
    wint = jnp.transpose(W_in)
    wcat = jnp.reshape(W_fb, (_NTAPS * _R, _R))
    bias2 = jnp.reshape(bias, (1, _R))
    states = _run_reservoir(xt2, wint, wcat, bias2)
    return jnp.swapaxes(jnp.reshape(states, (_S, _B, _R)), 0, 1)
